# baseline (speedup 1.0000x reference)
import functools

import jax
import jax.numpy as jnp
from jax import lax
from jax.experimental import pallas as pl
from jax.experimental.pallas import tpu as pltpu
from jax.experimental.pallas import tpu_sc as plsc

NTAGS = 16
NTOK = 16384
NROWS_GRP = 8
NC = 2
NS = 16
NW = NC * NS
BPW = NTOK // NW
CHUNK = 128
NCHUNK = BPW // CHUNK
NGRP = CHUNK // 16

_mesh = plsc.VectorSubcoreMesh(core_axis_name="c", subcore_axis_name="s")


@functools.partial(
    pl.kernel,
    out_type=(jax.ShapeDtypeStruct((NC, NTAGS), jnp.float32),
              jax.ShapeDtypeStruct((NC, NS, NTAGS), jnp.float32)),
    mesh=_mesh,
    scratch_types=[
        pltpu.VMEM((NCHUNK, CHUNK), jnp.int32),
        pltpu.VMEM((NCHUNK, CHUNK), jnp.int32),
        pltpu.VMEM((NCHUNK, CHUNK, 128), jnp.float32),
        pltpu.VMEM((NTAGS,), jnp.float32),
        pltpu.VMEM((NS, NTAGS), jnp.float32),
        pltpu.VMEM((NTAGS,), jnp.float32),
        pltpu.SemaphoreType.DMA,
    ],
)
def _bow_sc(x_hbm, grp_hbm, table_hbm, bias_hbm, out_hbm, scr_hbm,
            idx_v, grp_v, rows_v, acc_v, comb_v, bias_v, sem):
    cid = lax.axis_index("c")
    sid = lax.axis_index("s")
    wid = sid * NC + cid

    pltpu.sync_copy(x_hbm.at[wid], idx_v)
    pltpu.sync_copy(grp_hbm.at[wid], grp_v)

    copies = [
        pltpu.async_copy(table_hbm.at[grp_v.at[j]], rows_v.at[j], sem)
        for j in range(NCHUNK)
    ]
    for c in copies:
        c.wait()

    zero16 = jnp.zeros((16,), jnp.float32)

    def body(k, accs):
        out = []
        for j in range(NCHUNK):
            xv = idx_v[j, pl.ds(k * 16, 16)]
            a = accs[j]
            for l in range(16):
                sub = jnp.bitwise_and(xv[l], 7) * 16
                a = a + rows_v[j, k * 16 + l, pl.ds(sub, 16)]
            out.append(a)
        return tuple(out)

    accs = lax.fori_loop(0, NGRP, body,
                         tuple(zero16 for _ in range(NCHUNK)))
    acc_v[...] = (accs[0] + accs[1]) + (accs[2] + accs[3])

    pltpu.sync_copy(acc_v, scr_hbm.at[cid].at[sid])
    plsc.subcore_barrier()

    @pl.when(sid == 0)
    def _():
        pltpu.sync_copy(scr_hbm.at[cid], comb_v)
        pltpu.sync_copy(bias_hbm, bias_v)
        core_sum = comb_v[0, :]
        for t in range(1, NS):
            core_sum = core_sum + comb_v[t, :]

        @pl.when(cid == 0)
        def _():
            acc_v[...] = core_sum + bias_v[...]

        @pl.when(cid != 0)
        def _():
            acc_v[...] = core_sum

        pltpu.sync_copy(acc_v, out_hbm.at[cid])


def kernel(x, table, bias):
    x4 = x.reshape(NW, NCHUNK, CHUNK)
    g4 = lax.shift_right_logical(x4, 3)
    t2 = table.reshape(table.shape[0] // NROWS_GRP, NTAGS * NROWS_GRP)
    partials, _ = _bow_sc(x4, g4, t2, bias)
    return (partials[0] + partials[1]).reshape(1, NTAGS)


# speedup vs baseline: 1.0011x; 1.0011x over previous
import functools

import jax
import jax.numpy as jnp
from jax import lax
from jax.experimental import pallas as pl
from jax.experimental.pallas import tpu as pltpu
from jax.experimental.pallas import tpu_sc as plsc

NTAGS = 16
NTOK = 16384
NROWS_GRP = 8
NC = 2
NS = 16
NW = NC * NS
BPW = NTOK // NW
CHUNK = 128
NCHUNK = BPW // CHUNK
NGRP = CHUNK // 16

_mesh = plsc.VectorSubcoreMesh(core_axis_name="c", subcore_axis_name="s")


@functools.partial(
    pl.kernel,
    out_type=(jax.ShapeDtypeStruct((NC, NTAGS), jnp.float32),
              jax.ShapeDtypeStruct((NC, NS, NTAGS), jnp.float32)),
    mesh=_mesh,
    scratch_types=[
        pltpu.VMEM((NCHUNK, CHUNK), jnp.int32),
        pltpu.VMEM((NCHUNK, CHUNK), jnp.int32),
        pltpu.VMEM((NCHUNK, CHUNK, 128), jnp.float32),
        pltpu.VMEM((NTAGS,), jnp.float32),
        pltpu.VMEM((NS, NTAGS), jnp.float32),
        pltpu.VMEM((NTAGS,), jnp.float32),
        pltpu.SemaphoreType.DMA,
    ],
    compiler_params=pltpu.CompilerParams(use_tc_tiling_on_sc=True),
)
def _bow_sc(x_hbm, grp_hbm, table_hbm, bias_hbm, out_hbm, scr_hbm,
            idx_v, grp_v, rows_v, acc_v, comb_v, bias_v, sem):
    cid = lax.axis_index("c")
    sid = lax.axis_index("s")
    wid = sid * NC + cid

    pltpu.sync_copy(x_hbm.at[wid], idx_v)
    pltpu.sync_copy(grp_hbm.at[wid], grp_v)

    copies = [
        pltpu.async_copy(table_hbm.at[grp_v.at[j]], rows_v.at[j], sem)
        for j in range(NCHUNK)
    ]
    for c in copies:
        c.wait()

    zero16 = jnp.zeros((16,), jnp.float32)

    def body(k, accs):
        out = []
        for j in range(NCHUNK):
            xv = idx_v[j, pl.ds(k * 16, 16)]
            a = accs[j]
            for l in range(16):
                sub = jnp.bitwise_and(xv[l], 7) * 16
                a = a + rows_v[j, k * 16 + l, pl.ds(sub, 16)]
            out.append(a)
        return tuple(out)

    accs = lax.fori_loop(0, NGRP, body,
                         tuple(zero16 for _ in range(NCHUNK)))
    acc_v[...] = (accs[0] + accs[1]) + (accs[2] + accs[3])

    pltpu.sync_copy(acc_v, scr_hbm.at[cid].at[sid])
    plsc.subcore_barrier()

    @pl.when(sid == 0)
    def _():
        pltpu.sync_copy(scr_hbm.at[cid], comb_v)
        pltpu.sync_copy(bias_hbm, bias_v)
        core_sum = comb_v[0, :]
        for t in range(1, NS):
            core_sum = core_sum + comb_v[t, :]

        @pl.when(cid == 0)
        def _():
            acc_v[...] = core_sum + bias_v[...]

        @pl.when(cid != 0)
        def _():
            acc_v[...] = core_sum

        pltpu.sync_copy(acc_v, out_hbm.at[cid])


def kernel(x, table, bias):
    x4 = x.reshape(NW, NCHUNK, CHUNK)
    g4 = lax.shift_right_logical(x4, 3)
    t2 = table.reshape(table.shape[0] // NROWS_GRP, NTAGS * NROWS_GRP)
    partials, _ = _bow_sc(x4, g4, t2, bias)
    return (partials[0] + partials[1]).reshape(1, NTAGS)
